# Initial kernel scaffold; baseline (speedup 1.0000x reference)
import jax
import jax.numpy as jnp
from jax.experimental import pallas as pl


def kernel(x, params):
    raise NotImplementedError("write your pallas kernel here")


# speedup vs baseline: 1.1946x; 1.1946x over previous
import functools

import jax
import jax.numpy as jnp
from jax.experimental import pallas as pl
from jax.experimental.pallas import tpu as pltpu

_T = 2048
_E = 8
_NG = 4
_GS = _E // _NG


def _hdot(a, b, prec=None):
    return jax.lax.dot_general(
        a, b, (((1,), (1,)), ((), ())), precision=prec,
        preferred_element_type=jnp.float32)


def _bdot(a, b):
    return jax.lax.dot_general(
        a.astype(jnp.bfloat16), b.astype(jnp.bfloat16),
        (((1,), (1,)), ((), ())), preferred_element_type=jnp.float32)


def _silu(v):
    return v * jax.nn.sigmoid(v)


def _routing_combine(z, gate):
    logits = _hdot(z, gate)
    m = jnp.max(logits, axis=1, keepdims=True)
    p = jnp.exp(logits - m)
    scores = p / jnp.sum(p, axis=1, keepdims=True)
    lane = jax.lax.broadcasted_iota(jnp.int32, scores.shape, 1)
    cols = [
        jnp.sum(jnp.where(lane == j, scores, 0.0), axis=1, keepdims=True)
        for j in range(_E)
    ]
    gsc = [jnp.maximum(cols[2 * g], cols[2 * g + 1]) for g in range(_NG)]
    sel = []
    for g in range(_NG):
        rank = jnp.zeros_like(gsc[g])
        for j in range(_NG):
            if j == g:
                continue
            beats = (gsc[j] > gsc[g]) | ((gsc[j] == gsc[g]) & (j < g))
            rank = rank + beats.astype(jnp.float32)
        sel.append((rank < 2.0).astype(jnp.float32))
    group_lane = lane // _GS
    emask = jnp.zeros_like(scores)
    for g in range(_NG):
        emask = emask + jnp.where(group_lane == g, sel[g], 0.0)
    return scores * emask


def _small_layer_body(dout, x_ref, wl_ref, bl_ref, ws_ref, bs_ref, gate_ref,
                      w1_ref, w3_ref, w2_ref, s1_ref, s3_ref, s2_ref, out_ref):
    x = x_ref[...]
    y = _hdot(x, wl_ref[...]) + bl_ref[...]
    h = _hdot(y, ws_ref[...]) + bs_ref[...]
    u = h[:, :dout]
    v = h[:, dout:]
    z = u * _silu(v)
    combine = _routing_combine(z, gate_ref[...])
    a1 = _hdot(z, s1_ref[...])
    a3 = _hdot(z, s3_ref[...])
    acc = _hdot(_silu(a1) * a3, s2_ref[...])
    for e in range(_E):
        h1 = _hdot(z, w1_ref[e])
        h3 = _hdot(z, w3_ref[e])
        eo = _hdot(_silu(h1) * h3, w2_ref[e])
        w = jnp.sum(
            jnp.where(jax.lax.broadcasted_iota(jnp.int32, combine.shape, 1) == e,
                      combine, 0.0), axis=1, keepdims=True)
        acc = acc + w * eo
    out_ref[...] = acc


def _moeff_small(p, x, din, dout, tb=512):
    hdim = dout // 2
    hs = dout
    moe = p["moe"]
    full = lambda shp: pl.BlockSpec(shp, lambda i: (0,) * len(shp))
    grid = (_T // tb,)
    return pl.pallas_call(
        functools.partial(_small_layer_body, dout),
        grid=grid,
        in_specs=[
            pl.BlockSpec((tb, din), lambda i: (i, 0)),
            full((dout, din)), full((1, dout)),
            full((2 * dout, dout)), full((1, 2 * dout)),
            full((_E, dout)),
            full((_E, hdim, dout)), full((_E, hdim, dout)),
            full((_E, dout, hdim)),
            full((hs, dout)), full((hs, dout)), full((dout, hs)),
        ],
        out_specs=pl.BlockSpec((tb, dout), lambda i: (i, 0)),
        out_shape=jax.ShapeDtypeStruct((_T, dout), jnp.float32),
    )(x, p["lin"]["W"], p["lin"]["b"].reshape(1, -1),
      p["sw"]["W"], p["sw"]["b"].reshape(1, -1), moe["gate"],
      moe["w1"], moe["w3"], moe["w2"],
      moe["sw1"], moe["sw3"], moe["sw2"])


def _big_prelude_body(dout, x_ref, wl_ref, bl_ref, ws_ref, bs_ref, gate_ref,
                      s1_ref, s3_ref, s2_ref, z_ref, comb_ref, shared_ref):
    x = x_ref[...]
    y = _hdot(x, wl_ref[...]) + bl_ref[...]
    h = _hdot(y, ws_ref[...]) + bs_ref[...]
    u = h[:, :dout]
    v = h[:, dout:]
    z = u * _silu(v)
    comb_ref[...] = _routing_combine(z, gate_ref[...])
    zb = z.astype(jnp.bfloat16)
    z_ref[...] = zb
    a1 = _bdot(zb, s1_ref[...])
    a3 = _bdot(zb, s3_ref[...])
    shared_ref[...] = _bdot(_silu(a1) * a3, s2_ref[...])


def _big_routed_body(z_ref, comb_ref, shared_ref, w1_ref, w3_ref, w2_ref,
                     out_ref):
    e = pl.program_id(1)

    @pl.when(e == 0)
    def _():
        out_ref[...] = shared_ref[...]

    zb = z_ref[...]
    h1 = _bdot(zb, w1_ref[0])
    h3 = _bdot(zb, w3_ref[0])
    eo = _bdot(_silu(h1) * h3, w2_ref[0])
    comb = comb_ref[...]
    w = jnp.sum(
        jnp.where(jax.lax.broadcasted_iota(jnp.int32, comb.shape, 1) == e,
                  comb, 0.0), axis=1, keepdims=True)
    out_ref[...] = out_ref[...] + w * eo


def _moeff_big(p, x):
    din, dout, hdim = 512, 1536, 768
    hs = dout
    moe = p["moe"]
    full = lambda shp: pl.BlockSpec(shp, lambda i: (0,) * len(shp))
    tb = 256
    z, comb, shared = pl.pallas_call(
        functools.partial(_big_prelude_body, dout),
        grid=(_T // tb,),
        in_specs=[
            pl.BlockSpec((tb, din), lambda i: (i, 0)),
            full((dout, din)), full((1, dout)),
            full((2 * dout, dout)), full((1, 2 * dout)),
            full((_E, dout)),
            full((hs, dout)), full((hs, dout)), full((dout, hs)),
        ],
        out_specs=[
            pl.BlockSpec((tb, dout), lambda i: (i, 0)),
            pl.BlockSpec((tb, _E), lambda i: (i, 0)),
            pl.BlockSpec((tb, dout), lambda i: (i, 0)),
        ],
        out_shape=[
            jax.ShapeDtypeStruct((_T, dout), jnp.bfloat16),
            jax.ShapeDtypeStruct((_T, _E), jnp.float32),
            jax.ShapeDtypeStruct((_T, dout), jnp.float32),
        ],
    )(x, p["lin"]["W"], p["lin"]["b"].reshape(1, -1),
      p["sw"]["W"], p["sw"]["b"].reshape(1, -1), moe["gate"],
      moe["sw1"].astype(jnp.bfloat16), moe["sw3"].astype(jnp.bfloat16),
      moe["sw2"].astype(jnp.bfloat16))

    tbr = 1024
    out = pl.pallas_call(
        _big_routed_body,
        grid=(_T // tbr, _E),
        in_specs=[
            pl.BlockSpec((tbr, dout), lambda i, e: (i, 0)),
            pl.BlockSpec((tbr, _E), lambda i, e: (i, 0)),
            pl.BlockSpec((tbr, dout), lambda i, e: (i, 0)),
            pl.BlockSpec((1, hdim, dout), lambda i, e: (e, 0, 0)),
            pl.BlockSpec((1, hdim, dout), lambda i, e: (e, 0, 0)),
            pl.BlockSpec((1, dout, hdim), lambda i, e: (e, 0, 0)),
        ],
        out_specs=pl.BlockSpec((tbr, dout), lambda i, e: (i, 0)),
        out_shape=jax.ShapeDtypeStruct((_T, dout), jnp.float32),
    )(z, comb, shared,
      moe["w1"].astype(jnp.bfloat16), moe["w3"].astype(jnp.bfloat16),
      moe["w2"].astype(jnp.bfloat16))
    return out


def kernel(x, params):
    skip = _moeff_small(params["enc0"], x, 768, 256)
    b = _moeff_small(params["bott"], skip, 256, 256)
    d = jnp.concatenate([b, skip], axis=1)
    return _moeff_big(params["dec0"], d)
